# baseline (device time: 55163 ns/iter reference)
import jax
import jax.numpy as jnp
from jax import lax
from jax.experimental import pallas as pl
from jax.experimental.pallas import tpu as pltpu

_N_CHUNKS = 8


def kernel(x, pi):
    _, m, _ = x.shape
    rows = m // _N_CHUNKS

    def body(pi_ref, x_hbm, out_hbm, ld_buf, send_buf, ld_sems,
             send_sems, recv_sems):
        my_x = lax.axis_index("x")
        my_y = lax.axis_index("y")
        my_z = lax.axis_index("z")
        tgt = pi_ref[my_x]

        barrier = pltpu.get_barrier_semaphore()

        def load(k):
            return pltpu.make_async_copy(
                x_hbm.at[:, pl.ds(k * rows, rows), :],
                ld_buf.at[k % 2],
                ld_sems.at[k % 2],
            )

        @pl.when(tgt == my_x)
        def _():
            stores = []
            for k in range(_N_CHUNKS):
                sl = pl.ds(k * rows, rows)
                cp = load(k)
                cp.start()
                cp.wait()
                send_buf[0, sl, :] = ld_buf[k % 2, 0].astype(send_buf.dtype)
                st = pltpu.make_async_copy(
                    send_buf.at[:, sl, :], out_hbm.at[:, sl, :],
                    send_sems.at[k],
                )
                st.start()
                stores.append(st)
            for st in stores:
                st.wait()

        @pl.when(tgt != my_x)
        def _():
            pl.semaphore_signal(
                barrier,
                inc=1,
                device_id=(tgt, my_y, my_z),
                device_id_type=pl.DeviceIdType.MESH,
            )

            loads = {0: load(0), 1: load(1)}
            loads[0].start()
            loads[1].start()
            rdmas = []
            for k in range(_N_CHUNKS):
                sl = pl.ds(k * rows, rows)
                loads[k].wait()
                send_buf[0, sl, :] = ld_buf[k % 2, 0].astype(send_buf.dtype)
                if k + 2 < _N_CHUNKS:
                    loads[k + 2] = load(k + 2)
                    loads[k + 2].start()
                if k == 0:
                    pl.semaphore_wait(barrier, 1)
                rdma = pltpu.make_async_remote_copy(
                    src_ref=send_buf.at[:, sl, :],
                    dst_ref=out_hbm.at[:, sl, :],
                    send_sem=send_sems.at[k],
                    recv_sem=recv_sems.at[k],
                    device_id=(tgt, my_y, my_z),
                    device_id_type=pl.DeviceIdType.MESH,
                )
                rdma.start()
                rdmas.append(rdma)
            for rdma in rdmas:
                rdma.wait()

    return pl.pallas_call(
        body,
        out_shape=jax.ShapeDtypeStruct(x.shape, jnp.bfloat16),
        in_specs=[
            pl.BlockSpec(memory_space=pltpu.SMEM),
            pl.BlockSpec(memory_space=pl.ANY),
        ],
        out_specs=pl.BlockSpec(memory_space=pl.ANY),
        scratch_shapes=[
            pltpu.VMEM((2, 1, rows, x.shape[2]), x.dtype),
            pltpu.VMEM(x.shape, jnp.bfloat16),
            pltpu.SemaphoreType.DMA((2,)),
            pltpu.SemaphoreType.DMA((_N_CHUNKS,)),
            pltpu.SemaphoreType.DMA((_N_CHUNKS,)),
        ],
        compiler_params=pltpu.CompilerParams(collective_id=0),
    )(pi, x)


# device time: 54960 ns/iter; 1.0037x vs baseline; 1.0037x over previous
import jax
import jax.numpy as jnp
from jax import lax
from jax.experimental import pallas as pl
from jax.experimental.pallas import tpu as pltpu

_N_CHUNKS = 8


def kernel(x, pi):
    _, m, _ = x.shape
    rows = m // _N_CHUNKS

    def body(pi_ref, x_ref, out_ref, send_buf, send_sems, recv_sems):
        my_x = lax.axis_index("x")
        my_y = lax.axis_index("y")
        my_z = lax.axis_index("z")
        tgt = pi_ref[my_x]

        barrier = pltpu.get_barrier_semaphore()

        @pl.when(tgt == my_x)
        def _():
            out_ref[...] = x_ref[...].astype(out_ref.dtype)

        @pl.when(tgt != my_x)
        def _():
            pl.semaphore_signal(
                barrier,
                inc=1,
                device_id=(tgt, my_y, my_z),
                device_id_type=pl.DeviceIdType.MESH,
            )

            rdmas = []
            for k in range(_N_CHUNKS):
                sl = pl.ds(k * rows, rows)
                send_buf[0, sl, :] = x_ref[0, sl, :].astype(send_buf.dtype)
                if k == 0:
                    pl.semaphore_wait(barrier, 1)
                rdma = pltpu.make_async_remote_copy(
                    src_ref=send_buf.at[:, sl, :],
                    dst_ref=out_ref.at[:, sl, :],
                    send_sem=send_sems.at[k],
                    recv_sem=recv_sems.at[k],
                    device_id=(tgt, my_y, my_z),
                    device_id_type=pl.DeviceIdType.MESH,
                )
                rdma.start()
                rdmas.append(rdma)
            for rdma in rdmas:
                rdma.wait()

    return pl.pallas_call(
        body,
        out_shape=jax.ShapeDtypeStruct(x.shape, jnp.bfloat16),
        in_specs=[
            pl.BlockSpec(memory_space=pltpu.SMEM),
            pl.BlockSpec(memory_space=pltpu.VMEM),
        ],
        out_specs=pl.BlockSpec(memory_space=pltpu.VMEM),
        scratch_shapes=[
            pltpu.VMEM(x.shape, jnp.bfloat16),
            pltpu.SemaphoreType.DMA((_N_CHUNKS,)),
            pltpu.SemaphoreType.DMA((_N_CHUNKS,)),
        ],
        compiler_params=pltpu.CompilerParams(collective_id=0),
    )(pi, x)
